# baseline (speedup 1.0000x reference)
import functools

import numpy as np
import jax
from jax import lax
import jax.numpy as jnp
from jax.experimental import pallas as pl
from jax.experimental.pallas import tpu as pltpu
from jax.experimental.pallas import tpu_sc as plsc

_N_ATOMS = 4096
_N_MOLS = 128
_N_ELEM = 4
_NSTACKS = 32
_NPCAS = 128
_SIGMA = 3.0
_NFEAT = _NSTACKS * _NPCAS

_B = 256
_NBLK = _N_ATOMS // _B

_COEFF_NORM = np.float32(np.sqrt(np.float32(_NPCAS)) / _SIGMA)


def _hadamard(n):
    H = np.array([[1.0]], dtype=np.float64)
    while H.shape[0] < n:
        H = np.block([[H, H], [H, -H]])
    return H


def _select4(z_col, tbl):
    r01 = jnp.where(z_col == 0, tbl[0:1, :], tbl[1:2, :])
    r23 = jnp.where(z_col == 2, tbl[2:3, :], tbl[3:4, :])
    return jnp.where(z_col <= 1, r01, r23)


def _tc_body(rep_ref, d0_ref, d1_ref, bias_ref, alpha_ref, hn_ref,
             z_ref, e_ref):
    z = z_ref[0, 0, :].reshape(_B, 1)
    rep = rep_ref[...]

    zoh = (z == lax.broadcasted_iota(jnp.int32, (1, _N_ELEM), 1)
           ).astype(jnp.bfloat16)
    d0 = lax.dot(zoh, d0_ref[...],
                 preferred_element_type=jnp.float32)
    d1 = lax.dot(zoh, d1_ref[...], preferred_element_type=jnp.float32)
    b = _select4(z, bias_ref[...])

    hn = hn_ref[...]
    v = (rep[:, None, :] * d0.reshape(_B, _NSTACKS, _NPCAS)).reshape(
        _B * _NSTACKS, _NPCAS)
    v = lax.dot(v, hn)
    v = v * d1.reshape(_B * _NSTACKS, _NPCAS)
    v = lax.dot(v, hn)

    feats = jnp.cos(_COEFF_NORM * v.reshape(_B, _NFEAT) + b)
    e = jnp.sum(feats * alpha_ref[...], axis=1)
    e_ref[...] = e.reshape(1, 1, _B)


_NSUB = 16
_CHUNK = _N_ATOMS // _NSUB
_L = 16


def _sc_segsum(e_hbm, mol_hbm, out_hbm, e_v, mol_v, acc2_v, part_v, stage_v,
               shared):
    c = lax.axis_index("c")
    s = lax.axis_index("s")

    @pl.when(c == 0)
    def _():
        base = s * _CHUNK
        pltpu.sync_copy(e_hbm.at[pl.ds(base, _CHUNK)], e_v)
        pltpu.sync_copy(mol_hbm.at[pl.ds(base, _CHUNK)], mol_v)

        zero16 = jnp.zeros((_L,), jnp.float32)
        for j in range(_L * _N_MOLS // _L):
            acc2_v[pl.ds(j * _L, _L)] = zero16

        rowoff = lax.iota(jnp.int32, _L) * _N_MOLS

        def body(i, carry):
            ids = mol_v[pl.ds(i * _L, _L)]
            vals = e_v[pl.ds(i * _L, _L)]
            plsc.addupdate_scatter(acc2_v, [ids + rowoff], vals)
            return carry

        lax.fori_loop(0, _CHUNK // _L, body, 0)

        for k in range(_N_MOLS // _L):
            ssum = zero16
            for r in range(_L):
                ssum = ssum + acc2_v[pl.ds(r * _N_MOLS + k * _L, _L)]
            part_v[pl.ds(k * _L, _L)] = ssum

        pltpu.sync_copy(part_v, shared.at[s])
        plsc.subcore_barrier()

        @pl.when(s == 0)
        def _():
            pltpu.sync_copy(shared, stage_v)
            for k in range(_N_MOLS // _L):
                ssum2 = jnp.zeros((_L,), jnp.float32)
                for r in range(_NSUB):
                    ssum2 = ssum2 + stage_v[r, pl.ds(k * _L, _L)]
                part_v[pl.ds(k * _L, _L)] = ssum2
            pltpu.sync_copy(part_v, out_hbm)


_SC_SEGSUM_CACHE = []


def _get_sc_segsum():
    if not _SC_SEGSUM_CACHE:
        k = functools.partial(
            pl.kernel,
            mesh=plsc.VectorSubcoreMesh(core_axis_name="c",
                                        subcore_axis_name="s"),
            out_type=jax.ShapeDtypeStruct((_N_MOLS,), jnp.float32),
            scratch_types=[
                pltpu.VMEM((_CHUNK,), jnp.float32),
                pltpu.VMEM((_CHUNK,), jnp.int32),
                pltpu.VMEM((_L * _N_MOLS,), jnp.float32),
                pltpu.VMEM((_N_MOLS,), jnp.float32),
                pltpu.VMEM((_NSUB, _N_MOLS), jnp.float32),
                pltpu.VMEM_SHARED((_NSUB, _N_MOLS), jnp.float32),
            ],
            compiler_params=pltpu.CompilerParams(needs_layout_passes=False),
        )(_sc_segsum)
        _SC_SEGSUM_CACHE.append(k)
    return _SC_SEGSUM_CACHE[0]


def kernel(rep, Dmat, bias, alpha, Z, mol_ids):
    hn = jnp.asarray(_hadamard(_NPCAS) / np.sqrt(_NPCAS), dtype=jnp.float32)
    alpha_s = (alpha * np.float32(np.sqrt(2.0 / _NFEAT))).reshape(1, _NFEAT)

    d0 = Dmat[:, 0].reshape(_N_ELEM, _NFEAT).astype(jnp.bfloat16)
    d1 = Dmat[:, 1].reshape(_N_ELEM, _NFEAT).astype(jnp.bfloat16)
    z3 = Z.reshape(_NBLK, 1, _B)

    e = pl.pallas_call(
        _tc_body,
        grid=(_NBLK,),
        in_specs=[
            pl.BlockSpec((_B, _NPCAS), lambda i: (i, 0)),
            pl.BlockSpec((_N_ELEM, _NFEAT), lambda i: (0, 0)),
            pl.BlockSpec((_N_ELEM, _NFEAT), lambda i: (0, 0)),
            pl.BlockSpec((_N_ELEM, _NFEAT), lambda i: (0, 0)),
            pl.BlockSpec((1, _NFEAT), lambda i: (0, 0)),
            pl.BlockSpec((_NPCAS, _NPCAS), lambda i: (0, 0)),
            pl.BlockSpec((1, 1, _B), lambda i: (i, 0, 0)),
        ],
        out_specs=pl.BlockSpec((1, 1, _B), lambda i: (i, 0, 0)),
        out_shape=jax.ShapeDtypeStruct((_NBLK, 1, _B), jnp.float32),
        compiler_params=pltpu.CompilerParams(
            dimension_semantics=("arbitrary",),
        ),
    )(rep, d0, d1, bias, alpha_s, hn, z3)

    return _get_sc_segsum()(e.reshape(_N_ATOMS), mol_ids)
